# baseline (speedup 1.0000x reference)
import functools

import jax
import jax.numpy as jnp
from jax import lax
from jax.experimental import pallas as pl
from jax.experimental.pallas import tpu as pltpu
from jax.experimental.pallas import tpu_sc as plsc

B = 4096 * 200
D = 64
NC, NS = 2, 16
NW = NC * NS
B_PER_W = B // NW
CHUNK = 1024
N_CHUNKS = B_PER_W // CHUNK


@functools.partial(
    pl.kernel,
    mesh=plsc.VectorSubcoreMesh(core_axis_name="c", subcore_axis_name="s"),
    out_type=jax.ShapeDtypeStruct((B, D), jnp.float32),
    compiler_params=pltpu.CompilerParams(use_tc_tiling_on_sc=False),
    scratch_types=[
        pltpu.VMEM((CHUNK,), jnp.int32),
        pltpu.VMEM((CHUNK, D), jnp.float32),
        pltpu.SemaphoreType.DMA,
    ],
)
def _gather_rows(table_hbm, ids_hbm, out_hbm, idx_v, rows_v, sem):
    wid = lax.axis_index("s") * NC + lax.axis_index("c")
    base = wid * B_PER_W

    def chunk_body(i, carry):
        off = base + i * CHUNK
        pltpu.sync_copy(ids_hbm.at[pl.ds(off, CHUNK)], idx_v)
        pltpu.async_copy(table_hbm.at[idx_v], rows_v, sem).wait()
        pltpu.sync_copy(rows_v, out_hbm.at[pl.ds(off, CHUNK)])
        return carry

    lax.fori_loop(0, N_CHUNKS, chunk_body, 0)


def kernel(input_ids, weight):
    flat = input_ids.reshape(B)
    out = _gather_rows(weight, flat)
    return out.reshape(input_ids.shape + (D,))


# speedup vs baseline: 1.0152x; 1.0152x over previous
import functools

import jax
import jax.numpy as jnp
from jax import lax
from jax.experimental import pallas as pl
from jax.experimental.pallas import tpu as pltpu
from jax.experimental.pallas import tpu_sc as plsc

B = 4096 * 200
D = 64
NC, NS = 2, 16
NW = NC * NS
B_PER_W = B // NW
CHUNK = 800
N_CHUNKS = B_PER_W // CHUNK
NBUF = 2


@functools.partial(
    pl.kernel,
    mesh=plsc.VectorSubcoreMesh(core_axis_name="c", subcore_axis_name="s"),
    out_type=jax.ShapeDtypeStruct((B, D), jnp.float32),
    compiler_params=pltpu.CompilerParams(use_tc_tiling_on_sc=False),
    scratch_types=(
        [pltpu.VMEM((CHUNK,), jnp.int32) for _ in range(NBUF)]
        + [pltpu.VMEM((CHUNK, D), jnp.float32) for _ in range(NBUF)]
        + [pltpu.SemaphoreType.DMA for _ in range(3 * NBUF)]
    ),
)
def _gather_rows(table_hbm, ids_hbm, out_hbm, idx0, idx1, rows0, rows1,
                 si0, si1, sg0, sg1, so0, so1):
    idx = [idx0, idx1]
    rows = [rows0, rows1]
    s_idx = [si0, si1]
    s_gat = [sg0, sg1]
    s_out = [so0, so1]

    wid = lax.axis_index("s") * NC + lax.axis_index("c")
    base = wid * B_PER_W

    def off(i):
        return base + i * CHUNK

    idx_cp, gat_cp, out_cp = {}, {}, {}

    def start_idx(i):
        b = i % NBUF
        idx_cp[i] = pltpu.async_copy(
            ids_hbm.at[pl.ds(off(i), CHUNK)], idx[b], s_idx[b])

    def start_gat(i):
        b = i % NBUF
        gat_cp[i] = pltpu.async_copy(table_hbm.at[idx[b]], rows[b], s_gat[b])

    def start_out(i):
        b = i % NBUF
        out_cp[i] = pltpu.async_copy(
            rows[b], out_hbm.at[pl.ds(off(i), CHUNK)], s_out[b])

    start_idx(0)
    if N_CHUNKS > 1:
        start_idx(1)
    idx_cp[0].wait()
    start_gat(0)
    for i in range(N_CHUNKS):
        gat_cp[i].wait()
        start_out(i)
        if i + 2 < N_CHUNKS:
            start_idx(i + 2)
        if i + 1 < N_CHUNKS:
            idx_cp[i + 1].wait()
            if i >= 1:
                out_cp[i - 1].wait()
            start_gat(i + 1)
    if N_CHUNKS > 1:
        out_cp[N_CHUNKS - 2].wait()
    out_cp[N_CHUNKS - 1].wait()


def kernel(input_ids, weight):
    flat = input_ids.reshape(B)
    out = _gather_rows(weight, flat)
    return out.reshape(input_ids.shape + (D,))
